# baseline (speedup 1.0000x reference)
import functools

import jax
import jax.numpy as jnp
from jax import lax
from jax.experimental import pallas as pl
from jax.experimental.pallas import tpu as pltpu
from jax.experimental.pallas import tpu_sc as plsc

_CH = 16384

_GDNUMS = lax.GatherDimensionNumbers(
    offset_dims=(), collapsed_slice_dims=(0,), start_index_map=(0,))


def _shuffle(v, idx):
    return lax.gather(v, idx[:, None], dimension_numbers=_GDNUMS,
                      slice_sizes=(1,),
                      mode=lax.GatherScatterMode.PROMISE_IN_BOUNDS)


def _allsum16(v, idxs):
    for idx in idxs:
        v = v + _shuffle(v, idx)
    return v


def _compute_chunk(xv, mxv, mnv, idxs):
    @plsc.parallel_loop(0, _CH, step=32, unroll=4)
    def _max_blocks(b):
        v0 = xv[pl.ds(b, 16)]
        v1 = xv[pl.ds(b + 16, 16)]
        e0 = jnp.exp(v0)
        e1 = jnp.exp(v1)
        re = 1.0 / _allsum16(e0 + e1, idxs)
        mxv[pl.ds(b, 16)] = e0 * re
        mxv[pl.ds(b + 16, 16)] = e1 * re

    @plsc.parallel_loop(0, _CH, step=32, unroll=4)
    def _min_blocks(b):
        v0 = xv[pl.ds(b, 16)]
        v1 = xv[pl.ds(b + 16, 16)]
        g0 = jnp.exp(-v0)
        g1 = jnp.exp(-v1)
        rg = 1.0 / _allsum16(g0 + g1, idxs)
        mnv[pl.ds(b, 16)] = g0 * rg
        mnv[pl.ds(b + 16, 16)] = g1 * rg


def _sc_body(x_hbm, out_hbm,
             xv0, xv1, mxv0, mxv1, mnv0, mnv1, si0, si1, so0, so1):
    w = lax.axis_index("s") * 2 + lax.axis_index("c")
    nch = x_hbm.shape[1] // _CH
    lanes = lax.iota(jnp.int32, 16)
    idxs = [lanes ^ (1 << k) for k in range(4)]
    bufs = ((xv0, mxv0, mnv0, si0, so0), (xv1, mxv1, mnv1, si1, so1))

    pltpu.async_copy(x_hbm.at[w, pl.ds(0, _CH)], xv0, si0)
    pltpu.async_copy(x_hbm.at[w, pl.ds(_CH, _CH)], xv1, si1)

    def pair(i, carry):
        k0 = i * 2
        for b in range(2):
            k = k0 + b
            xv, mxv, mnv, si, so = bufs[b]
            off = k * _CH
            pltpu.make_async_copy(
                x_hbm.at[w, pl.ds(off, _CH)], xv, si).wait()

            @pl.when(k >= 2)
            def _():
                pltpu.make_async_copy(
                    mxv, out_hbm.at[w, 0, pl.ds(0, _CH)], so).wait()
                pltpu.make_async_copy(
                    mnv, out_hbm.at[w, 1, pl.ds(0, _CH)], so).wait()

            _compute_chunk(xv, mxv, mnv, idxs)

            pltpu.async_copy(mxv, out_hbm.at[w, 0, pl.ds(off, _CH)], so)
            pltpu.async_copy(mnv, out_hbm.at[w, 1, pl.ds(off, _CH)], so)

            @pl.when(k + 2 < nch)
            def _():
                pltpu.async_copy(
                    x_hbm.at[w, pl.ds(off + 2 * _CH, _CH)], xv, si)
        return carry

    lax.fori_loop(0, nch // 2, pair, 0)

    for xv, mxv, mnv, si, so in bufs:
        pltpu.make_async_copy(mxv, out_hbm.at[w, 0, pl.ds(0, _CH)], so).wait()
        pltpu.make_async_copy(mnv, out_hbm.at[w, 1, pl.ds(0, _CH)], so).wait()


@jax.jit
def kernel(x):
    B, H, S, D = x.shape
    BH = B * H
    SEC = S * D
    x2 = x.reshape(BH, SEC)
    mesh = plsc.VectorSubcoreMesh(core_axis_name="c", subcore_axis_name="s")
    f = pl.kernel(
        _sc_body,
        out_type=jax.ShapeDtypeStruct((BH, 2, SEC), jnp.float32),
        mesh=mesh,
        scratch_types=(
            [pltpu.VMEM((_CH,), jnp.float32) for _ in range(6)]
            + [pltpu.SemaphoreType.DMA for _ in range(4)]
        ),
    )
    out = f(x2)
    return out.reshape(B, H, 2 * SEC)


# speedup vs baseline: 1.0154x; 1.0154x over previous
import jax
import jax.numpy as jnp
from jax import lax
from jax.experimental import pallas as pl
from jax.experimental.pallas import tpu as pltpu
from jax.experimental.pallas import tpu_sc as plsc

_CH = 16384

_GDNUMS = lax.GatherDimensionNumbers(
    offset_dims=(), collapsed_slice_dims=(0,), start_index_map=(0,))


def _shuffle(v, idx):
    return lax.gather(v, idx[:, None], dimension_numbers=_GDNUMS,
                      slice_sizes=(1,),
                      mode=lax.GatherScatterMode.PROMISE_IN_BOUNDS)


def _allsum16(v, idxs):
    for idx in idxs:
        v = v + _shuffle(v, idx)
    return v


def _compute_chunk(xv, mxv, mnv, idxs):
    @plsc.parallel_loop(0, _CH, step=32, unroll=3)
    def _blocks(b):
        v0 = xv[pl.ds(b, 16)]
        v1 = xv[pl.ds(b + 16, 16)]
        e0 = jnp.exp(v0)
        e1 = jnp.exp(v1)
        g0 = jnp.exp(-v0)
        g1 = jnp.exp(-v1)
        re = 1.0 / _allsum16(e0 + e1, idxs)
        rg = 1.0 / _allsum16(g0 + g1, idxs)
        mxv[pl.ds(b, 16)] = e0 * re
        mxv[pl.ds(b + 16, 16)] = e1 * re
        mnv[pl.ds(b, 16)] = g0 * rg
        mnv[pl.ds(b + 16, 16)] = g1 * rg


def _sc_body(x_hbm, out_hbm,
             xv0, xv1, mxv0, mxv1, mnv0, mnv1, si0, si1, so0, so1):
    w = lax.axis_index("s") * 2 + lax.axis_index("c")
    nch = x_hbm.shape[1] // _CH
    lanes = lax.iota(jnp.int32, 16)
    idxs = [lanes ^ (1 << k) for k in range(4)]
    bufs = ((xv0, mxv0, mnv0, si0, so0), (xv1, mxv1, mnv1, si1, so1))

    pltpu.async_copy(x_hbm.at[w, pl.ds(0, _CH)], xv0, si0)
    pltpu.async_copy(x_hbm.at[w, pl.ds(_CH, _CH)], xv1, si1)

    def pair(i, carry):
        k0 = i * 2
        for b in range(2):
            k = k0 + b
            xv, mxv, mnv, si, so = bufs[b]
            off = k * _CH
            pltpu.make_async_copy(
                x_hbm.at[w, pl.ds(off, _CH)], xv, si).wait()

            @pl.when(k >= 2)
            def _():
                pltpu.make_async_copy(
                    mxv, out_hbm.at[w, 0, pl.ds(0, _CH)], so).wait()
                pltpu.make_async_copy(
                    mnv, out_hbm.at[w, 1, pl.ds(0, _CH)], so).wait()

            _compute_chunk(xv, mxv, mnv, idxs)

            pltpu.async_copy(mxv, out_hbm.at[w, 0, pl.ds(off, _CH)], so)
            pltpu.async_copy(mnv, out_hbm.at[w, 1, pl.ds(off, _CH)], so)

            @pl.when(k + 2 < nch)
            def _():
                pltpu.async_copy(
                    x_hbm.at[w, pl.ds(off + 2 * _CH, _CH)], xv, si)
        return carry

    lax.fori_loop(0, nch // 2, pair, 0)

    for xv, mxv, mnv, si, so in bufs:
        pltpu.make_async_copy(mxv, out_hbm.at[w, 0, pl.ds(0, _CH)], so).wait()
        pltpu.make_async_copy(mnv, out_hbm.at[w, 1, pl.ds(0, _CH)], so).wait()


@jax.jit
def kernel(x):
    B, H, S, D = x.shape
    BH = B * H
    SEC = S * D
    x2 = x.reshape(BH, SEC)
    mesh = plsc.VectorSubcoreMesh(core_axis_name="c", subcore_axis_name="s")
    f = pl.kernel(
        _sc_body,
        out_type=jax.ShapeDtypeStruct((BH, 2, SEC), jnp.float32),
        mesh=mesh,
        scratch_types=(
            [pltpu.VMEM((_CH,), jnp.float32) for _ in range(6)]
            + [pltpu.SemaphoreType.DMA for _ in range(4)]
        ),
    )
    out = f(x2)
    return out.reshape(B, H, 2 * SEC)
